# baseline (speedup 1.0000x reference)
import functools

import jax
import jax.numpy as jnp
from jax import lax
from jax.experimental import pallas as pl
from jax.experimental.pallas import tpu as pltpu
from jax.experimental.pallas import tpu_sc as plsc

OUT_H = 7
OUT_W = 7
SPATIAL_SCALE = 0.25
P_BINS = OUT_H * OUT_W
NC = 2
NS = 16
NW = NC * NS
BINS_PER_CHUNK = 7
N_CHUNKS = 7
ROWS_PER_BIN = 16
CHUNK_ROWS = BINS_PER_CHUNK * ROWS_PER_BIN


def _build_sc_kernel(n_img, h, w, c, rois_per_w, r_pad):
    hw = h * w
    cw = c // 2
    mesh = plsc.VectorSubcoreMesh(core_axis_name="c", subcore_axis_name="s")

    @functools.partial(
        pl.kernel,
        out_type=jax.ShapeDtypeStruct((r_pad, c * P_BINS), jnp.float32),
        mesh=mesh,
        compiler_params=pltpu.CompilerParams(needs_layout_passes=False),
        scratch_types=[
            pltpu.VMEM((rois_per_w + 8, 16), jnp.float32),
            pltpu.VMEM((2 * N_CHUNKS * CHUNK_ROWS,), jnp.int32),
            pltpu.VMEM((2 * N_CHUNKS * CHUNK_ROWS,), jnp.float32),
            pltpu.VMEM((CHUNK_ROWS, cw), jnp.int32),
            pltpu.VMEM((CHUNK_ROWS, cw), jnp.int32),
            pltpu.VMEM((CHUNK_ROWS, cw), jnp.int32),
            pltpu.VMEM((c * P_BINS,), jnp.float32),
            pltpu.SemaphoreType.DMA,
            pltpu.SemaphoreType.DMA,
            pltpu.SemaphoreType.DMA,
        ],
    )
    def sc_roi_align(table_hbm, rois_hbm, out_hbm, roi_v, idx_v, wgt_v,
                     buf0, buf1, buf2, out_v, sem0, sem1, sem2):
        cid = lax.axis_index("c")
        sid = lax.axis_index("s")
        wid = sid * NC + cid
        pltpu.sync_copy(
            rois_hbm.at[pl.ds(wid * rois_per_w, rois_per_w + 8)], roi_v)

        lane = lax.iota(jnp.int32, 16)
        samp = lax.shift_right_logical(lane, 2)
        corner = lane & 3
        sy_off = (lax.shift_right_logical(samp, 1).astype(jnp.float32)
                  + 0.5) * 0.5
        sx_off = ((samp & 1).astype(jnp.float32) + 0.5) * 0.5
        is_yh = (corner & 2) == 2
        is_xh = (corner & 1) == 1
        lane_p = lane * P_BINS
        fh = jnp.float32(h - 1)
        fw = jnp.float32(w - 1)
        hi_mask = jnp.int32(-65536)

        def phase1(ri, sl):
            rv = roi_v[ri, pl.ds(0, 16)]
            bb = rv[0]
            x1 = rv[1] * SPATIAL_SCALE
            y1 = rv[2] * SPATIAL_SCALE
            x2 = rv[3] * SPATIAL_SCALE
            y2 = rv[4] * SPATIAL_SCALE
            roi_w = jnp.maximum(x2 - x1, 1.0)
            roi_h = jnp.maximum(y2 - y1, 1.0)
            bin_w = roi_w * jnp.float32(1.0 / OUT_W)
            bin_h = roi_h * jnp.float32(1.0 / OUT_H)
            base = bb.astype(jnp.int32) * hw

            def chunk_idx_body(ch, carry2):
                phf = ch.astype(jnp.float32)
                for b in range(BINS_PER_CHUNK):
                    pwf = jnp.float32(b)
                    y = y1 + (phf + sy_off) * bin_h
                    x = x1 + (pwf + sx_off) * bin_w
                    m = ((y >= -1.0) & (y <= jnp.float32(h))
                         & (x >= -1.0) & (x <= jnp.float32(w)))
                    yc = jnp.maximum(y, 0.0)
                    xc = jnp.maximum(x, 0.0)
                    yl0 = yc.astype(jnp.int32)
                    xl0 = xc.astype(jnp.int32)
                    cy = yl0 >= h - 1
                    cx = xl0 >= w - 1
                    yl = jnp.where(cy, h - 1, yl0)
                    yhi = jnp.where(cy, h - 1, yl0 + 1)
                    yv = jnp.where(cy, fh, yc)
                    xl = jnp.where(cx, w - 1, xl0)
                    xhi = jnp.where(cx, w - 1, xl0 + 1)
                    xv = jnp.where(cx, fw, xc)
                    ly = yv - yl.astype(jnp.float32)
                    hy = 1.0 - ly
                    lx = xv - xl.astype(jnp.float32)
                    hx = 1.0 - lx
                    wy = jnp.where(is_yh, ly, hy)
                    wx = jnp.where(is_xh, lx, hx)
                    wgt = wy * wx * jnp.where(m, jnp.float32(0.25),
                                              jnp.float32(0.0))
                    ysel = jnp.where(is_yh, yhi, yl)
                    xsel = jnp.where(is_xh, xhi, xl)
                    idx = base + ysel * w + xsel
                    off = pl.multiple_of(
                        sl * (N_CHUNKS * CHUNK_ROWS) + ch * CHUNK_ROWS
                        + b * 16, 16)
                    idx_v[pl.ds(off, 16)] = idx
                    wgt_v[pl.ds(off, 16)] = wgt
                return carry2

            lax.fori_loop(0, N_CHUNKS, chunk_idx_body, 0)

        def idx_ref(sl, chk):
            off = pl.multiple_of(
                sl * (N_CHUNKS * CHUNK_ROWS) + chk * CHUNK_ROWS, 16)
            return idx_v.at[pl.ds(off, CHUNK_ROWS)]

        def start(sl, chk, bufr, sem):
            return pltpu.async_copy(table_hbm.at[idx_ref(sl, chk)],
                                    bufr, sem)

        def wait_c0(sl):
            pltpu.make_async_copy(table_hbm.at[idx_ref(sl, 0)],
                                  buf0, sem0).wait()

        def acc_chunk(chk, sl, bufr):
            @plsc.parallel_loop(0, BINS_PER_CHUNK)
            def acc_body(b):
                p = chk * BINS_PER_CHUNK + b
                woff = pl.multiple_of(
                    sl * (N_CHUNKS * CHUNK_ROWS) + chk * CHUNK_ROWS
                    + b * ROWS_PER_BIN, 16)
                wvec = wgt_v[pl.ds(woff, ROWS_PER_BIN)]
                acc_e = [jnp.zeros((16,), jnp.float32) for _ in range(8)]
                acc_o = [jnp.zeros((16,), jnp.float32) for _ in range(8)]
                for j in range(ROWS_PER_BIN):
                    row = b * ROWS_PER_BIN + j
                    wj = wvec[j]
                    for k in range(8):
                        word = bufr[row, pl.ds(k * 16, 16)]
                        ev = plsc.bitcast(lax.shift_left(word, 16),
                                          jnp.float32)
                        od = plsc.bitcast(word & hi_mask, jnp.float32)
                        acc_e[k] = acc_e[k] + wj * ev
                        acc_o[k] = acc_o[k] + wj * od
                for k in range(8):
                    plsc.store_scatter(
                        out_v, [lane_p + (p + k * 16 * P_BINS)], acc_e[k])
                    plsc.store_scatter(
                        out_v, [lane_p + (p + (128 + k * 16) * P_BINS)],
                        acc_o[k])

        phase1(0, 0)
        start(0, 0, buf0, sem0)

        def roi_body(i, carry):
            sl = i & 1
            nsl = 1 - sl
            descs = {}
            descs[1] = start(sl, 1, buf1, sem1)
            descs[2] = start(sl, 2, buf2, sem2)
            wait_c0(sl)
            acc_chunk(0, sl, buf0)
            phase1(i + 1, nsl)
            start(nsl, 0, buf0, sem0)
            bufs = {1: buf1, 2: buf2}
            semss = {1: sem1, 2: sem2}
            for chk in range(1, N_CHUNKS):
                par = 1 + ((chk - 1) % 2)
                descs[chk].wait()
                acc_chunk(chk, sl, bufs[par])
                if chk + 2 < N_CHUNKS:
                    descs[chk + 2] = start(sl, chk + 2, bufs[par], semss[par])
            pltpu.sync_copy(out_v, out_hbm.at[wid + i * NW])
            return carry

        lax.fori_loop(0, rois_per_w, roi_body, 0)
        wait_c0(0)

    return sc_roi_align


def kernel(features, rois):
    n_img, c, h, w = features.shape
    r = rois.shape[0]
    rois_per_w = -(-r // NW)
    r_pad = rois_per_w * NW
    def _rne_bf16_bits(x):
        u = lax.bitcast_convert_type(x, jnp.int32)
        return lax.shift_right_logical(
            u + 0x7FFF + (lax.shift_right_logical(u, 16) & 1), 16)
    lo = _rne_bf16_bits(features[:, :c // 2])
    hi = _rne_bf16_bits(features[:, c // 2:])
    packed = lo | lax.shift_left(hi, 16)
    table = (jnp.transpose(packed, (0, 2, 3, 1))
             .reshape(n_img * h * w, c // 2))
    rpad = (jnp.pad(rois, ((0, r_pad - r), (0, 11)))
            .reshape(rois_per_w, NW, 16).transpose(1, 0, 2)
            .reshape(r_pad, 16))
    rpad = jnp.pad(rpad, ((0, 8), (0, 0)))
    sc_fn = _build_sc_kernel(n_img, h, w, c, rois_per_w, r_pad)
    out = sc_fn(table, rpad)
    return out[:r].reshape(r, c, OUT_H, OUT_W)


# speedup vs baseline: 1.7082x; 1.7082x over previous
import functools

import jax
import jax.numpy as jnp
from jax import lax
from jax.experimental import pallas as pl
from jax.experimental.pallas import tpu as pltpu
from jax.experimental.pallas import tpu_sc as plsc

OUT_H = 7
OUT_W = 7
SPATIAL_SCALE = 0.25
P_BINS = OUT_H * OUT_W
NC = 2
NS = 16
NW = NC * NS
BINS_PER_CHUNK = 7
N_CHUNKS = 7
ROWS_PER_BIN = 16
CHUNK_ROWS = BINS_PER_CHUNK * ROWS_PER_BIN


def _build_sc_kernel(n_img, h, w, c, rois_per_w, r_pad):
    hw = h * w
    cw = c // 2
    mesh = plsc.VectorSubcoreMesh(core_axis_name="c", subcore_axis_name="s")

    @functools.partial(
        pl.kernel,
        out_type=jax.ShapeDtypeStruct((r_pad, c * P_BINS), jnp.float32),
        mesh=mesh,
        compiler_params=pltpu.CompilerParams(needs_layout_passes=False),
        scratch_types=[
            pltpu.VMEM((rois_per_w + 8, 16), jnp.float32),
            pltpu.VMEM((2 * N_CHUNKS * CHUNK_ROWS,), jnp.int32),
            pltpu.VMEM((2 * N_CHUNKS * CHUNK_ROWS,), jnp.float32),
            pltpu.VMEM((CHUNK_ROWS, cw), jnp.int32),
            pltpu.VMEM((CHUNK_ROWS, cw), jnp.int32),
            pltpu.VMEM((CHUNK_ROWS, cw), jnp.int32),
            pltpu.VMEM((c * P_BINS,), jnp.float32),
            pltpu.SemaphoreType.DMA,
            pltpu.SemaphoreType.DMA,
            pltpu.SemaphoreType.DMA,
        ],
    )
    def sc_roi_align(table_hbm, rois_hbm, out_hbm, roi_v, idx_v, wgt_v,
                     buf0, buf1, buf2, out_v, sem0, sem1, sem2):
        cid = lax.axis_index("c")
        sid = lax.axis_index("s")
        wid = sid * NC + cid
        pltpu.sync_copy(
            rois_hbm.at[pl.ds(wid * rois_per_w, rois_per_w + 8)], roi_v)

        lane = lax.iota(jnp.int32, 16)
        samp = lax.shift_right_logical(lane, 2)
        corner = lane & 3
        sy_off = (lax.shift_right_logical(samp, 1).astype(jnp.float32)
                  + 0.5) * 0.5
        sx_off = ((samp & 1).astype(jnp.float32) + 0.5) * 0.5
        is_yh = (corner & 2) == 2
        is_xh = (corner & 1) == 1
        lane_p = lane * P_BINS
        fh = jnp.float32(h - 1)
        fw = jnp.float32(w - 1)
        hi_mask = jnp.int32(-65536)

        def phase1(ri, sl):
            rv = roi_v[ri, pl.ds(0, 16)]
            bb = rv[0]
            x1 = rv[1] * SPATIAL_SCALE
            y1 = rv[2] * SPATIAL_SCALE
            x2 = rv[3] * SPATIAL_SCALE
            y2 = rv[4] * SPATIAL_SCALE
            roi_w = jnp.maximum(x2 - x1, 1.0)
            roi_h = jnp.maximum(y2 - y1, 1.0)
            bin_w = roi_w * jnp.float32(1.0 / OUT_W)
            bin_h = roi_h * jnp.float32(1.0 / OUT_H)
            base = bb.astype(jnp.int32) * hw

            def chunk_idx_body(ch, carry2):
                phf = ch.astype(jnp.float32)
                for b in range(BINS_PER_CHUNK):
                    pwf = jnp.float32(b)
                    y = y1 + (phf + sy_off) * bin_h
                    x = x1 + (pwf + sx_off) * bin_w
                    m = ((y >= -1.0) & (y <= jnp.float32(h))
                         & (x >= -1.0) & (x <= jnp.float32(w)))
                    yc = jnp.maximum(y, 0.0)
                    xc = jnp.maximum(x, 0.0)
                    yl0 = yc.astype(jnp.int32)
                    xl0 = xc.astype(jnp.int32)
                    cy = yl0 >= h - 1
                    cx = xl0 >= w - 1
                    yl = jnp.where(cy, h - 1, yl0)
                    yhi = jnp.where(cy, h - 1, yl0 + 1)
                    yv = jnp.where(cy, fh, yc)
                    xl = jnp.where(cx, w - 1, xl0)
                    xhi = jnp.where(cx, w - 1, xl0 + 1)
                    xv = jnp.where(cx, fw, xc)
                    ly = yv - yl.astype(jnp.float32)
                    hy = 1.0 - ly
                    lx = xv - xl.astype(jnp.float32)
                    hx = 1.0 - lx
                    wy = jnp.where(is_yh, ly, hy)
                    wx = jnp.where(is_xh, lx, hx)
                    wgt = wy * wx * jnp.where(m, jnp.float32(0.25),
                                              jnp.float32(0.0))
                    ysel = jnp.where(is_yh, yhi, yl)
                    xsel = jnp.where(is_xh, xhi, xl)
                    idx = base + ysel * w + xsel
                    off = pl.multiple_of(
                        sl * (N_CHUNKS * CHUNK_ROWS) + ch * CHUNK_ROWS
                        + b * 16, 16)
                    idx_v[pl.ds(off, 16)] = idx
                    wgt_v[pl.ds(off, 16)] = wgt
                return carry2

            lax.fori_loop(0, N_CHUNKS, chunk_idx_body, 0)

        def idx_ref(sl, chk):
            off = pl.multiple_of(
                sl * (N_CHUNKS * CHUNK_ROWS) + chk * CHUNK_ROWS, 16)
            return idx_v.at[pl.ds(off, CHUNK_ROWS)]

        def start(sl, chk, bufr, sem):
            return pltpu.async_copy(table_hbm.at[idx_ref(sl, chk)],
                                    bufr, sem)

        def wait_c0(sl):
            pltpu.make_async_copy(table_hbm.at[idx_ref(sl, 0)],
                                  buf0, sem0).wait()

        def acc_chunk(chk, sl, bufr):
            def acc_body(b, carry3):
                p = chk * BINS_PER_CHUNK + b
                woff = pl.multiple_of(
                    sl * (N_CHUNKS * CHUNK_ROWS) + chk * CHUNK_ROWS
                    + b * ROWS_PER_BIN, 16)
                wvec = wgt_v[pl.ds(woff, ROWS_PER_BIN)]
                acc_e = [jnp.zeros((16,), jnp.float32) for _ in range(8)]
                acc_o = [jnp.zeros((16,), jnp.float32) for _ in range(8)]
                for j in range(ROWS_PER_BIN):
                    row = b * ROWS_PER_BIN + j
                    wj = wvec[j]
                    for k in range(8):
                        word = bufr[row, pl.ds(k * 16, 16)]
                        ev = plsc.bitcast(lax.shift_left(word, 16),
                                          jnp.float32)
                        od = plsc.bitcast(word & hi_mask, jnp.float32)
                        acc_e[k] = acc_e[k] + wj * ev
                        acc_o[k] = acc_o[k] + wj * od
                for k in range(8):
                    plsc.store_scatter(
                        out_v, [lane_p + (p + k * 16 * P_BINS)], acc_e[k])
                    plsc.store_scatter(
                        out_v, [lane_p + (p + (128 + k * 16) * P_BINS)],
                        acc_o[k])
                return carry3

            lax.fori_loop(0, BINS_PER_CHUNK, acc_body, 0)

        phase1(0, 0)
        start(0, 0, buf0, sem0)

        def roi_body(i, carry):
            sl = i & 1
            nsl = 1 - sl
            descs = {}
            descs[1] = start(sl, 1, buf1, sem1)
            descs[2] = start(sl, 2, buf2, sem2)
            wait_c0(sl)
            acc_chunk(0, sl, buf0)
            phase1(i + 1, nsl)
            start(nsl, 0, buf0, sem0)
            bufs = {1: buf1, 2: buf2}
            semss = {1: sem1, 2: sem2}
            for chk in range(1, N_CHUNKS):
                par = 1 + ((chk - 1) % 2)
                descs[chk].wait()
                acc_chunk(chk, sl, bufs[par])
                if chk + 2 < N_CHUNKS:
                    descs[chk + 2] = start(sl, chk + 2, bufs[par], semss[par])
            pltpu.sync_copy(out_v, out_hbm.at[wid + i * NW])
            return carry

        lax.fori_loop(0, rois_per_w, roi_body, 0)
        wait_c0(0)

    return sc_roi_align


def kernel(features, rois):
    n_img, c, h, w = features.shape
    r = rois.shape[0]
    rois_per_w = -(-r // NW)
    r_pad = rois_per_w * NW
    def _rne_bf16_bits(x):
        u = lax.bitcast_convert_type(x, jnp.int32)
        return lax.shift_right_logical(
            u + 0x7FFF + (lax.shift_right_logical(u, 16) & 1), 16)
    lo = _rne_bf16_bits(features[:, :c // 2])
    hi = _rne_bf16_bits(features[:, c // 2:])
    packed = lo | lax.shift_left(hi, 16)
    table = (jnp.transpose(packed, (0, 2, 3, 1))
             .reshape(n_img * h * w, c // 2))
    rpad = (jnp.pad(rois, ((0, r_pad - r), (0, 11)))
            .reshape(rois_per_w, NW, 16).transpose(1, 0, 2)
            .reshape(r_pad, 16))
    rpad = jnp.pad(rpad, ((0, 8), (0, 0)))
    sc_fn = _build_sc_kernel(n_img, h, w, c, rois_per_w, r_pad)
    out = sc_fn(table, rpad)
    return out[:r].reshape(r, c, OUT_H, OUT_W)
